# baseline (speedup 1.0000x reference)
import functools

import jax
import jax.numpy as jnp
from jax import lax
from jax.experimental import pallas as pl

_N = 16
_H = 4
_OMEGA0 = 30.0
_CIN = 64
_COUT = 64
_A = 16.0


def _moe_block_kernel(x_ref, c_ref, w_ref, o_ref):
    xb = x_ref[...]
    cb = c_ref[...]
    wcat = w_ref[...]

    affine = cb * _A
    xi = jnp.floor(affine[:, 0]).astype(jnp.int32) % _H
    yi = jnp.floor(affine[:, 1]).astype(jnp.int32) % _H
    tid = _H * xi + yi

    y = jnp.dot(xb, wcat, preferred_element_type=jnp.float32)

    bt = xb.shape[0]
    col_expert = lax.broadcasted_iota(jnp.int32, (bt, _N * _COUT), 1) // _COUT
    mask = col_expert == tid[:, None]
    z = jnp.where(mask, y, 0.0)
    acc = z.reshape(bt, _N, _COUT).sum(axis=1)
    o_ref[...] = jnp.sin(_OMEGA0 * acc)


@jax.jit
def kernel(in_feats, in_coords, W):
    B = in_feats.shape[0]
    bt = 512
    coords = in_coords.reshape(B, 2)
    wcat = jnp.transpose(W.reshape(_N * _COUT, _CIN))

    out = pl.pallas_call(
        _moe_block_kernel,
        grid=(B // bt,),
        in_specs=[
            pl.BlockSpec((bt, _CIN), lambda i: (i, 0)),
            pl.BlockSpec((bt, 2), lambda i: (i, 0)),
            pl.BlockSpec((_CIN, _N * _COUT), lambda i: (0, 0)),
        ],
        out_specs=pl.BlockSpec((bt, _COUT), lambda i: (i, 0)),
        out_shape=jax.ShapeDtypeStruct((B, _COUT), jnp.float32),
    )(in_feats, coords, wcat)
    return out


# speedup vs baseline: 4.4468x; 4.4468x over previous
import functools

import jax
import jax.numpy as jnp
from jax import lax
from jax.experimental import pallas as pl

_N = 16
_H = 4
_OMEGA0 = 30.0
_CIN = 64
_COUT = 64
_A = 16.0


def _moe_block_kernel(x_ref, c_ref, w_ref, s_ref, o_ref):
    xb = x_ref[...]
    cb = c_ref[...]
    wcat = w_ref[...]
    sel = s_ref[...]

    affine = cb * _A
    xi = jnp.floor(affine[:, 0]).astype(jnp.int32) % _H
    yi = jnp.floor(affine[:, 1]).astype(jnp.int32) % _H
    tid = _H * xi + yi

    y = jnp.dot(xb, wcat, preferred_element_type=jnp.float32)

    bt = xb.shape[0]
    col_expert = lax.broadcasted_iota(jnp.int32, (bt, _N * _COUT), 1) // _COUT
    mask = col_expert == tid[:, None]
    z = jnp.where(mask, y, 0.0)
    acc = jnp.dot(z, sel, preferred_element_type=jnp.float32)
    o_ref[...] = jnp.sin(_OMEGA0 * acc)


@jax.jit
def kernel(in_feats, in_coords, W):
    B = in_feats.shape[0]
    bt = 512
    coords = in_coords.reshape(B, 2)
    wcat = jnp.transpose(W.reshape(_N * _COUT, _CIN))
    sel = jnp.tile(jnp.eye(_COUT, dtype=jnp.float32), (_N, 1))

    out = pl.pallas_call(
        _moe_block_kernel,
        grid=(B // bt,),
        in_specs=[
            pl.BlockSpec((bt, _CIN), lambda i: (i, 0)),
            pl.BlockSpec((bt, 2), lambda i: (i, 0)),
            pl.BlockSpec((_CIN, _N * _COUT), lambda i: (0, 0)),
            pl.BlockSpec((_N * _COUT, _COUT), lambda i: (0, 0)),
        ],
        out_specs=pl.BlockSpec((bt, _COUT), lambda i: (i, 0)),
        out_shape=jax.ShapeDtypeStruct((B, _COUT), jnp.float32),
    )(in_feats, coords, wcat, sel)
    return out
